# baseline (speedup 1.0000x reference)
import functools

import jax
import jax.numpy as jnp
from jax import lax
from jax.experimental import pallas as pl
from jax.experimental.pallas import tpu as pltpu
from jax.experimental.pallas import tpu_sc as plsc

B = 4096
N = 1000
NP = B * N
NW = 32
ROWS_W = B // NW
CB = 8
NCHUNK = ROWS_W // CB
CH = CB * N
GROUPS = CH // 16
NPAD = 1024

_CLS_COEF = 0.002
_OFF_COEF = 0.004


def _sc_body(cls_hbm, po_hbm, oe_hbm, c1_out, part_out,
             clsbuf, pobuf, oebuf, c1buf, accbuf):
    wid = lax.axis_index("s") * 2 + lax.axis_index("c")
    b0 = wid * ROWS_W

    iota = lax.broadcasted_iota(jnp.int32, (16,), 0)
    zeros_i = jnp.zeros((16,), jnp.int32)
    ones_i = jnp.ones((16,), jnp.int32)

    acc = jnp.zeros((16,), jnp.float32)
    for c in range(NCHUNK):
        bc = b0 + c * CB
        k0 = bc * N
        pltpu.sync_copy(cls_hbm.at[pl.ds(2 * k0, 2 * CH)], clsbuf)
        pltpu.sync_copy(po_hbm.at[pl.ds(2 * k0, 2 * CH)], pobuf)
        pltpu.sync_copy(oe_hbm.at[pl.ds(2 * k0, 2 * CH)], oebuf)

        def body(j, a):
            idx = j * 16 + iota
            idx2 = 2 * idx
            c0 = plsc.load_gather(clsbuf, [idx2])
            c1 = plsc.load_gather(clsbuf, [idx2 + 1])
            p0 = plsc.load_gather(pobuf, [idx2])
            p1 = plsc.load_gather(pobuf, [idx2 + 1])
            e0 = plsc.load_gather(oebuf, [idx2])
            e1 = plsc.load_gather(oebuf, [idx2 + 1])
            m = c1 > c0
            zf = jnp.zeros((16,), jnp.float32)
            a = (a + jnp.abs(p0 - jnp.where(m, e0, zf))
                 + jnp.abs(p1 - jnp.where(m, e1, zf)))
            b_loc = lax.shift_right_logical(idx * 8389, 23)
            plsc.store_scatter(c1buf, [idx + 24 * b_loc], c1)
            return a

        acc = lax.fori_loop(0, GROUPS, body, acc)
        pltpu.sync_copy(c1buf, c1_out.at[pl.ds(bc * NPAD, CB * NPAD)])

    accbuf[...] = acc
    pltpu.sync_copy(accbuf, part_out.at[pl.ds(wid * 16, 16)])


_sc_deinterleave = functools.partial(
    pl.kernel,
    mesh=plsc.VectorSubcoreMesh(core_axis_name="c", subcore_axis_name="s"),
    out_type=[
        jax.ShapeDtypeStruct((B * NPAD,), jnp.float32),
        jax.ShapeDtypeStruct((NW * 16,), jnp.float32),
    ],
    scratch_types=[
        pltpu.VMEM((2 * CH,), jnp.float32),
        pltpu.VMEM((2 * CH,), jnp.float32),
        pltpu.VMEM((2 * CH,), jnp.float32),
        pltpu.VMEM((CB * NPAD,), jnp.float32),
        pltpu.VMEM((16,), jnp.float32),
    ],
    compiler_params=pltpu.CompilerParams(needs_layout_passes=False,
                                         use_tc_tiling_on_sc=False),
)(_sc_body)

BMY = 128
TCG = B // BMY


def _tc_kernel(c1_ref, y_ref, part_ref, out_ref):
    i = pl.program_id(0)

    cv = c1_ref[...]
    cw = cv.reshape(BMY, NPAD)[:, :N]
    yv = y_ref[...]

    ls = -(jnp.maximum(-cw, 0.0) + jnp.log1p(jnp.exp(-jnp.abs(cw))))
    bce = (1.0 - yv) * cw - (1.0 + 9.0 * yv) * ls
    part = _CLS_COEF * jnp.sum(bce)

    @pl.when(i == 0)
    def _init():
        out_ref[...] = jnp.full(
            (1, 1), _OFF_COEF * jnp.sum(part_ref[...]), jnp.float32)

    out_ref[...] += jnp.full((1, 1), part, dtype=jnp.float32)


def _tc_loss(c1m, y, parts):
    out = pl.pallas_call(
        _tc_kernel,
        grid=(TCG,),
        in_specs=[
            pl.BlockSpec((8 * BMY, 128), lambda i: (i, 0)),
            pl.BlockSpec((BMY, N), lambda i: (i, 0)),
            pl.BlockSpec((4, 128), lambda i: (0, 0)),
        ],
        out_specs=pl.BlockSpec((1, 1), lambda i: (0, 0)),
        out_shape=jax.ShapeDtypeStruct((1, 1), jnp.float32),
        compiler_params=pltpu.CompilerParams(
            dimension_semantics=("arbitrary",),
        ),
    )(c1m, y, parts)
    return out[0, 0]


def kernel(pred_target_prob, pred_offset, pred_RCNN_cls, gt_target_prob,
           gt_candidate, gt_offset_each, gt_target_candidate_lens):
    cls_h = pltpu.with_memory_space_constraint(
        pred_RCNN_cls.reshape(-1), pltpu.HBM)
    po_h = pltpu.with_memory_space_constraint(
        pred_offset.reshape(-1), pltpu.HBM)
    oe_h = pltpu.with_memory_space_constraint(
        gt_offset_each.reshape(-1), pltpu.HBM)
    c1flat, parts = _sc_deinterleave(cls_h, po_h, oe_h)
    c1m = c1flat.reshape(B * NPAD // 128, 128)
    return _tc_loss(c1m, gt_target_prob, parts.reshape(4, 128))


# speedup vs baseline: 2.8044x; 2.8044x over previous
import jax
import jax.numpy as jnp
from jax.experimental import pallas as pl
from jax.experimental.pallas import tpu as pltpu

B = 4096
N = 1000
BM = 128
GRID = B // BM

_CLS_COEF = 0.002
_OFF_COEF = 0.004


def _loss_kernel(cls_ref, y_ref, po_ref, oe_ref, out_ref):
    i = pl.program_id(0)

    z = cls_ref[...]
    yv = y_ref[...]
    po = po_ref[...]
    oe = oe_ref[...]

    parity = jax.lax.broadcasted_iota(jnp.int32, (BM, 2 * N), 1) % 2
    is_odd = parity == 1

    z_next = pltpu.roll(z, 2 * N - 1, 1)
    cmp = (z_next < z).astype(jnp.float32)
    cmp_r = pltpu.roll(cmp, 1, 1)
    pos = jnp.where(is_odd, cmp_r, cmp)

    off_term = jnp.abs(po - oe * pos)

    y2 = pltpu.repeat(yv, 2, 1)
    ls = -(jnp.maximum(z, 0.0) + jnp.log1p(jnp.exp(-jnp.abs(z))))
    bce = -(1.0 - y2) * z - (1.0 + 9.0 * y2) * ls
    bce = jnp.where(is_odd, bce, 0.0)

    part = _CLS_COEF * jnp.sum(bce) + _OFF_COEF * jnp.sum(off_term)
    part = jnp.full((1, 1), part, dtype=jnp.float32)

    @pl.when(i == 0)
    def _init():
        out_ref[...] = jnp.zeros((1, 1), jnp.float32)

    out_ref[...] += part


def _tnt_loss(cls2, y, po2, oe2):
    out = pl.pallas_call(
        _loss_kernel,
        grid=(GRID,),
        in_specs=[
            pl.BlockSpec((BM, 2 * N), lambda i: (i, 0)),
            pl.BlockSpec((BM, N), lambda i: (i, 0)),
            pl.BlockSpec((BM, 2 * N), lambda i: (i, 0)),
            pl.BlockSpec((BM, 2 * N), lambda i: (i, 0)),
        ],
        out_specs=pl.BlockSpec((1, 1), lambda i: (0, 0)),
        out_shape=jax.ShapeDtypeStruct((1, 1), jnp.float32),
        compiler_params=pltpu.CompilerParams(
            dimension_semantics=("arbitrary",),
            allow_input_fusion=[True, True, True, True],
        ),
    )(cls2, y, po2, oe2)
    return out[0, 0]


def kernel(pred_target_prob, pred_offset, pred_RCNN_cls, gt_target_prob,
           gt_candidate, gt_offset_each, gt_target_candidate_lens):
    ncls = -pred_RCNN_cls.reshape(B, 2 * N)
    npo = -pred_offset.reshape(B, 2 * N)
    noe = -gt_offset_each.reshape(B, 2 * N)
    return _tnt_loss(ncls, gt_target_prob, npo, noe)


# speedup vs baseline: 16.0697x; 5.7302x over previous
import jax
import jax.numpy as jnp
from jax.experimental import pallas as pl
from jax.experimental.pallas import tpu as pltpu

B = 4096
N = 1000
BM = 256
GRID = B // BM

_CLS_COEF = 0.002
_OFF_COEF = 0.004


def _loss_kernel(c0_ref, c1_ref, y_ref, p0_ref, p1_ref, e0_ref, e1_ref,
                 out_ref):
    i = pl.program_id(0)

    c0 = c0_ref[...]
    c1 = c1_ref[...]
    yv = y_ref[...]

    pos = (c1 > c0).astype(jnp.float32)
    off = (jnp.abs(p0_ref[...] - e0_ref[...] * pos)
           + jnp.abs(p1_ref[...] - e1_ref[...] * pos))

    ls = -(jnp.maximum(-c1, 0.0) + jnp.log1p(jnp.exp(-jnp.abs(c1))))
    bce = (1.0 - yv) * c1 - (1.0 + 9.0 * yv) * ls

    part = _CLS_COEF * jnp.sum(bce) + _OFF_COEF * jnp.sum(off)
    part = jnp.full((1, 1), part, dtype=jnp.float32)

    @pl.when(i == 0)
    def _init():
        out_ref[...] = jnp.zeros((1, 1), jnp.float32)

    out_ref[...] += part


def _tnt_loss(c0, c1, y, p0, p1, e0, e1):
    spec = pl.BlockSpec((BM, N), lambda i: (i, 0))
    out = pl.pallas_call(
        _loss_kernel,
        grid=(GRID,),
        in_specs=[spec] * 7,
        out_specs=pl.BlockSpec((1, 1), lambda i: (0, 0)),
        out_shape=jax.ShapeDtypeStruct((1, 1), jnp.float32),
        compiler_params=pltpu.CompilerParams(
            dimension_semantics=("arbitrary",),
            allow_input_fusion=[True] * 7,
        ),
    )(c0, c1, y, p0, p1, e0, e1)
    return out[0, 0]


def kernel(pred_target_prob, pred_offset, pred_RCNN_cls, gt_target_prob,
           gt_candidate, gt_offset_each, gt_target_candidate_lens):
    oe3 = gt_offset_each.reshape(B, N, 2)
    c0 = pred_RCNN_cls[:, :, 0]
    c1 = pred_RCNN_cls[:, :, 1]
    p0 = pred_offset[:, :, 0]
    p1 = pred_offset[:, :, 1]
    e0 = oe3[:, :, 0]
    e1 = oe3[:, :, 1]
    return _tnt_loss(c0, c1, gt_target_prob, p0, p1, e0, e1)
